# baseline (speedup 1.0000x reference)
import jax
import jax.numpy as jnp
import numpy as np
from jax.experimental import pallas as pl
from jax.experimental.pallas import tpu as pltpu

B = 128
D = 64
V = 100000
TILE = 4096
GRID = (V + TILE - 1) // TILE

_K0 = np.uint32(0)
_K1 = np.uint32(42)
_K2 = np.uint32(int(_K0) ^ int(_K1) ^ 0x1BD11BDA)
_ROT = ((13, 15, 26, 6), (17, 29, 16, 24))
_TINY = np.float32(np.finfo(np.float32).tiny)


def _rotl(x, r):
    return (x << np.uint32(r)) | (x >> np.uint32(32 - r))


def _threefry_bits(lo):
    ks = (_K0, _K1, _K2)
    x0 = jnp.full_like(lo, ks[0])
    x1 = lo + ks[1]
    for d in range(5):
        for r in _ROT[d % 2]:
            x0 = x0 + x1
            x1 = _rotl(x1, r)
            x1 = x1 ^ x0
        x0 = x0 + ks[(d + 1) % 3]
        x1 = x1 + ks[(d + 2) % 3] + np.uint32(d + 1)
    return x0 ^ x1


def _body(x_ref, w_ref, b_ref, out_ref, best_val, best_idx):
    i = pl.program_id(0)
    logits = jnp.dot(x_ref[...], w_ref[...], preferred_element_type=jnp.float32)
    logits = logits + b_ref[...]

    jglob = i * TILE + jax.lax.broadcasted_iota(jnp.int32, (B, TILE), 1)
    row = jax.lax.broadcasted_iota(jnp.int32, (B, TILE), 0)
    n = (row * V + jglob).astype(jnp.uint32)
    bits = _threefry_bits(n)

    fbits = (bits >> np.uint32(9)) | np.uint32(0x3F800000)
    floats = jax.lax.bitcast_convert_type(fbits, jnp.float32) - np.float32(1.0)
    span = np.float32(1.0) - _TINY
    u = jnp.maximum(_TINY, floats * span + _TINY)
    g = -jnp.log(-jnp.log(u))

    y = jnp.where(jglob < V, g + logits, -jnp.inf)
    m = jnp.max(y, axis=1, keepdims=True)
    idx = jnp.min(jnp.where(y == m, jglob, jnp.int32(2**31 - 1)),
                  axis=1, keepdims=True)

    @pl.when(i == 0)
    def _():
        best_val[...] = m
        best_idx[...] = idx

    @pl.when(i > 0)
    def _():
        better = m > best_val[...]
        best_val[...] = jnp.where(better, m, best_val[...])
        best_idx[...] = jnp.where(better, idx, best_idx[...])

    @pl.when(i == GRID - 1)
    def _():
        out_ref[...] = best_idx[...]


def kernel(inputs, W, b):
    b2d = b.reshape(1, V)
    sample = pl.pallas_call(
        _body,
        grid=(GRID,),
        in_specs=[
            pl.BlockSpec((B, D), lambda i: (0, 0)),
            pl.BlockSpec((D, TILE), lambda i: (0, i)),
            pl.BlockSpec((1, TILE), lambda i: (0, i)),
        ],
        out_specs=pl.BlockSpec((B, 1), lambda i: (0, 0)),
        out_shape=jax.ShapeDtypeStruct((B, 1), jnp.int32),
        scratch_shapes=[
            pltpu.VMEM((B, 1), jnp.float32),
            pltpu.VMEM((B, 1), jnp.int32),
        ],
        compiler_params=pltpu.CompilerParams(
            dimension_semantics=("arbitrary",)),
    )(inputs, W, b2d)
    ps = jnp.full((B,), 1.0 / B, dtype=jnp.float32)
    return (sample.reshape(B), ps)
